# baseline (speedup 1.0000x reference)
import functools

import jax
import jax.numpy as jnp
from jax.experimental import pallas as pl
from jax.experimental.pallas import tpu as pltpu

B = 16384
C = 10
HB = 8192
NSTAGE = B // HB


def _body(tp_ref, pp_ref, bayes_ref, o_ref, t_buf, p_buf, bayes_buf, sems):
    copies = []
    for h in range(NSTAGE):
        copies.append((
            pltpu.make_async_copy(
                tp_ref.at[:, :, pl.ds(h * HB, HB)], t_buf.at[h], sems.at[h, 0]),
            pltpu.make_async_copy(
                pp_ref.at[:, pl.ds(h * HB, HB)], p_buf.at[h], sems.at[h, 1]),
            pltpu.make_async_copy(
                bayes_ref.at[pl.ds(h * HB, HB)], bayes_buf.at[h], sems.at[h, 2]),
        ))
        for cp in copies[h]:
            cp.start()

    part = jnp.zeros((1, 1), jnp.float32)
    for h in range(NSTAGE):
        for cp in copies[h]:
            cp.wait()

        bayes = bayes_buf[h]
        m = [(bayes == r).astype(jnp.float32) for r in range(C)]

        acc_neg = jnp.zeros((HB,), jnp.float32)
        total = jnp.zeros((HB,), jnp.float32)
        for c in range(C):
            tc = t_buf.at[h][:, c, :]
            q_c = tc[0] * m[0]
            for r in range(1, C):
                q_c = q_c + tc[r] * m[r]
            sgn = 1.0 - 2.0 * p_buf[h, c, :].astype(jnp.float32)
            e_c = jnp.exp(q_c * sgn)
            acc_neg = acc_neg + e_c * (0.5 * (1.0 + sgn))
            total = total + e_c

        acc_pos = total - acc_neg
        part += jnp.sum(
            jnp.log1p(acc_neg * acc_pos), keepdims=True
        ).reshape(1, 1) * (1.0 / B)

    o_ref[...] = part


@jax.jit
def kernel(T, bayes, partial):
    tp = jnp.transpose(T, (1, 2, 0))
    pp = partial.T
    out = pl.pallas_call(
        _body,
        in_specs=[
            pl.BlockSpec(memory_space=pl.ANY),
            pl.BlockSpec(memory_space=pl.ANY),
            pl.BlockSpec(memory_space=pl.ANY),
        ],
        out_specs=pl.BlockSpec(memory_space=pltpu.VMEM),
        out_shape=jax.ShapeDtypeStruct((1, 1), jnp.float32),
        scratch_shapes=[
            pltpu.VMEM((NSTAGE, C, C, HB), jnp.float32),
            pltpu.VMEM((NSTAGE, C, HB), jnp.int32),
            pltpu.VMEM((NSTAGE, HB), jnp.int32),
            pltpu.SemaphoreType.DMA((NSTAGE, 3)),
        ],
    )(tp, pp, bayes)
    return out[0, 0]
